# baseline (speedup 1.0000x reference)
import functools

import jax
import jax.numpy as jnp
from jax import lax
from jax.experimental import pallas as pl
from jax.experimental.pallas import tpu as pltpu
from jax.experimental.pallas import tpu_sc as plsc

N = 10000
E = 160000
EP = 163840
D_IN = 128
HID = 144
NB = 10
MAX_R = 5.0
INV_SQRT_NUM_NEIGH = 0.25
SH = 9
NP = 10240
BE = 512
CH = 128


@functools.lru_cache(maxsize=None)
def _mesh():
    return plsc.VectorSubcoreMesh(
        core_axis_name="c", subcore_axis_name="s", num_cores=2, num_subcores=16)


def _sc_gather(table, idx_flat, sc_tiling=False):
    rows = idx_flat.shape[0]
    width = table.shape[1]
    dt = table.dtype
    rowbytes = width * dt.itemsize
    ch = 128 if rowbytes <= 512 else 64
    U = 4
    nch = rows // (32 * ch)
    idxr = idx_flat.reshape(rows // ch, ch)
    cp = (pltpu.CompilerParams(use_tc_tiling_on_sc=False) if sc_tiling
          else None)

    @functools.partial(
        pl.kernel,
        out_type=jax.ShapeDtypeStruct((rows, width), dt),
        mesh=_mesh(),
        scratch_types=[
            pltpu.VMEM((nch, ch), jnp.int32),
            pltpu.VMEM((U * ch, width), dt),
        ] + [pltpu.SemaphoreType.DMA] * (2 * U),
        compiler_params=cp,
    )
    def k(table_hbm, idx_hbm, out_hbm, idx_v, bufs, *sems):
        gsem, wsem = sems[:U], sems[U:]
        wid = lax.axis_index("s") * 2 + lax.axis_index("c")
        pltpu.sync_copy(idx_hbm.at[pl.ds(wid * nch, nch)], idx_v)
        base = wid * nch * ch

        def body(t, carry):
            for u in range(U):
                j = t * U + u
                slot = bufs.at[pl.ds(u * ch, ch)]

                @pl.when(t > 0)
                def _(j=j, slot=slot, u=u):
                    pltpu.make_async_copy(
                        slot, out_hbm.at[pl.ds(base + (j - U) * ch, ch)],
                        wsem[u]).wait()

                pltpu.async_copy(table_hbm.at[idx_v.at[j]], slot, gsem[u])
            for u in range(U):
                j = t * U + u
                slot = bufs.at[pl.ds(u * ch, ch)]
                pltpu.make_async_copy(
                    table_hbm.at[idx_v.at[j]], slot, gsem[u]).wait()
                pltpu.async_copy(
                    slot, out_hbm.at[pl.ds(base + j * ch, ch)], wsem[u])
            return carry

        lax.fori_loop(0, nch // U, body, 0)
        for u in range(U):
            j = nch - U + u
            pltpu.make_async_copy(
                bufs.at[pl.ds(u * ch, ch)],
                out_hbm.at[pl.ds(base + j * ch, ch)], wsem[u]).wait()

    return k(table, idxr)


def _sc_scatter(pa, pb, idx2d, zeros_init):
    nch = EP // (16 * CH)
    rps = NP // 16
    U = 2

    @functools.partial(
        pl.kernel,
        out_type=jax.ShapeDtypeStruct((2 * NP, 128), jnp.float32),
        mesh=_mesh(),
        scratch_types=[
            pltpu.VMEM((nch, CH), jnp.int32),
            pltpu.VMEM((U * CH, 128), jnp.float32),
            pltpu.VMEM_SHARED((NP, 128), jnp.float32),
        ] + [pltpu.SemaphoreType.DMA] * (2 * U),
    )
    def k(pa_hbm, pb_hbm, idx_hbm, z_hbm, out_hbm, idx_v, bufs, acc, *sems):
        rsem, ssem = sems[:U], sems[U:]
        cid = lax.axis_index("c")
        sid = lax.axis_index("s")
        pltpu.sync_copy(z_hbm.at[pl.ds(sid * rps, rps)],
                        acc.at[pl.ds(sid * rps, rps)])
        plsc.subcore_barrier()
        pltpu.sync_copy(idx_hbm.at[pl.ds(sid * nch, nch)], idx_v)
        base = sid * nch * CH

        for p_hbm, c in ((pa_hbm, 0), (pb_hbm, 1)):
            @pl.when(cid == c)
            def _(p_hbm=p_hbm):
                def body(t, carry):
                    for u in range(U):
                        j = t * U + u
                        slot = bufs.at[pl.ds(u * CH, CH)]

                        @pl.when(t > 0)
                        def _(j=j, slot=slot, u=u):
                            pltpu.make_async_copy(
                                slot, acc.at[idx_v.at[j - U]], ssem[u]).wait()

                        pltpu.async_copy(
                            p_hbm.at[pl.ds(base + j * CH, CH)], slot, rsem[u])
                    for u in range(U):
                        j = t * U + u
                        slot = bufs.at[pl.ds(u * CH, CH)]
                        pltpu.make_async_copy(
                            p_hbm.at[pl.ds(base + j * CH, CH)], slot,
                            rsem[u]).wait()
                        pltpu.async_copy(slot, acc.at[idx_v.at[j]], ssem[u],
                                         add=True)
                    return carry

                lax.fori_loop(0, nch // U, body, 0)
                for u in range(U):
                    j = nch - U + u
                    pltpu.make_async_copy(
                        bufs.at[pl.ds(u * CH, CH)], acc.at[idx_v.at[j]],
                        ssem[u]).wait()

        plsc.subcore_barrier()
        pltpu.sync_copy(acc.at[pl.ds(sid * rps, rps)],
                        out_hbm.at[pl.ds(cid * NP + sid * rps, rps)])

    return k(pa, pb, idx2d, zeros_init)


def _tc_geometry(poss, posd, shift_p, lat_p):
    pw = poss.shape[1]

    def body(ps_ref, pd_ref, sf_ref, lat_ref, f_ref):
        i = pl.program_id(0)
        shmm = jnp.dot(sf_ref[...], lat_ref[...],
                       preferred_element_type=jnp.float32)
        vs = pd_ref[:, :16] - ps_ref[:, :16]
        vx = vs[:, 0:1] + shmm[:, 0:1]
        vy = vs[:, 1:2] + shmm[:, 1:2]
        vz = vs[:, 2:3] + shmm[:, 2:3]
        r = jnp.sqrt(vx * vx + vy * vy + vz * vz + 1e-9)
        xx, yy, zz = vx / r, vy / r, vz / r
        s3 = 3.0 ** 0.5
        s15 = 15.0 ** 0.5
        sh = jnp.concatenate([
            jnp.ones_like(xx), s3 * xx, s3 * yy, s3 * zz,
            s15 * xx * yy, s15 * yy * zz,
            ((5.0 ** 0.5) / 2.0) * (3.0 * zz * zz - 1.0),
            s15 * xx * zz, (s15 / 2.0) * (xx * xx - yy * yy),
        ], axis=1)
        eidx = i * BE + lax.broadcasted_iota(jnp.int32, (BE, 1), 0)
        sh = sh * (eidx < E).astype(jnp.float32)
        step = MAX_R / (NB + 1)
        vals = (lax.broadcasted_iota(jnp.int32, (1, NB), 1).astype(jnp.float32)
                + 1.0) * step
        diff = (r - vals) / step
        emb = (jnp.cos(jnp.pi / 2.0 * diff)
               * ((diff > -1.0) & (diff < 1.0)).astype(jnp.float32)
               * (NB ** 0.5))
        f_ref[...] = jnp.concatenate(
            [sh, emb, jnp.zeros((BE, 32 - SH - NB), jnp.float32)], axis=1)

    return pl.pallas_call(
        body,
        grid=(EP // BE,),
        in_specs=[
            pl.BlockSpec((BE, pw), lambda i: (i, 0)),
            pl.BlockSpec((BE, pw), lambda i: (i, 0)),
            pl.BlockSpec((BE, 16), lambda i: (i, 0)),
            pl.BlockSpec((16, 16), lambda i: (0, 0)),
        ],
        out_specs=pl.BlockSpec((BE, 32), lambda i: (i, 0)),
        out_shape=jax.ShapeDtypeStruct((EP, 32), jnp.float32),
    )(poss, posd, shift_p, lat_p)


def _tc_layer(F, hsrc, w1p, b1, w2, lin, di):
    tw = hsrc.shape[1]
    lin_hi = lin.astype(jnp.bfloat16)
    lin_lo = (lin - lin_hi.astype(jnp.float32)).astype(jnp.bfloat16)

    def body(f_ref, h_ref, w1_ref, b1_ref, w2_ref, lhi_ref, llo_ref,
             pa_ref, pb_ref):
        hid = jnp.maximum(
            jnp.dot(f_ref[...], w1_ref[...], preferred_element_type=jnp.float32)
            + b1_ref[...], 0.0)
        w = jnp.dot(hid, w2_ref[...], preferred_element_type=jnp.float32)
        m = w * h_ref[:, :di].astype(jnp.float32)
        m_hi = m.astype(jnp.bfloat16)
        m_lo = (m - m_hi.astype(jnp.float32)).astype(jnp.bfloat16)
        acc = jnp.zeros((BE, HID), jnp.float32)
        for k in range(SH):
            sl = slice(k * di, (k + 1) * di)
            q = (jnp.dot(m_hi, lhi_ref[sl, :], preferred_element_type=jnp.float32)
                 + jnp.dot(m_hi, llo_ref[sl, :], preferred_element_type=jnp.float32)
                 + jnp.dot(m_lo, lhi_ref[sl, :], preferred_element_type=jnp.float32))
            acc = acc + f_ref[:, k:k + 1] * q
        pa_ref[...] = acc[:, :128]
        pb_ref[...] = jnp.concatenate(
            [acc[:, 128:], jnp.zeros((BE, 256 - HID), jnp.float32)], axis=1)

    return pl.pallas_call(
        body,
        grid=(EP // BE,),
        in_specs=[
            pl.BlockSpec((BE, 32), lambda i: (i, 0)),
            pl.BlockSpec((BE, tw), lambda i: (i, 0)),
            pl.BlockSpec((32, 100), lambda i: (0, 0)),
            pl.BlockSpec((1, 100), lambda i: (0, 0)),
            pl.BlockSpec((100, di), lambda i: (0, 0)),
            pl.BlockSpec((SH * di, HID), lambda i: (0, 0)),
            pl.BlockSpec((SH * di, HID), lambda i: (0, 0)),
        ],
        out_specs=[
            pl.BlockSpec((BE, 128), lambda i: (i, 0)),
            pl.BlockSpec((BE, 128), lambda i: (i, 0)),
        ],
        out_shape=[
            jax.ShapeDtypeStruct((EP, 128), jnp.float32),
            jax.ShapeDtypeStruct((EP, 128), jnp.float32),
        ],
    )(F, hsrc, w1p, b1, w2, lin_hi, lin_lo)


def _tc_node(p0, p1, h, sc, di):
    bn = 1000
    hw = h.shape[1]

    def body(p0_ref, p1_ref, h_ref, sc_ref, o_ref, of_ref, cs_ref):
        pre = jnp.concatenate(
            [p0_ref[...], p1_ref[:, :HID - 128]],
            axis=1) * INV_SQRT_NUM_NEIGH + jnp.dot(
            h_ref[:, :di], sc_ref[...], preferred_element_type=jnp.float32)
        hn = jax.nn.gelu(pre)
        o_ref[...] = jnp.concatenate(
            [hn, jnp.zeros((bn, 256 - HID), jnp.float32)], axis=1)
        of_ref[...] = hn

        @pl.when(pl.program_id(0) == 0)
        def _():
            cs_ref[...] = jnp.zeros_like(cs_ref)

        cs_ref[...] += jnp.sum(hn, axis=0, keepdims=True)

    return pl.pallas_call(
        body,
        grid=(N // bn,),
        in_specs=[
            pl.BlockSpec((bn, 128), lambda i: (i, 0)),
            pl.BlockSpec((bn, 128), lambda i: (i, 0)),
            pl.BlockSpec((bn, hw), lambda i: (i, 0)),
            pl.BlockSpec((di, HID), lambda i: (0, 0)),
        ],
        out_specs=[
            pl.BlockSpec((bn, 256), lambda i: (i, 0)),
            pl.BlockSpec((bn, HID), lambda i: (i, 0)),
            pl.BlockSpec((1, HID), lambda i: (0, 0)),
        ],
        out_shape=[
            jax.ShapeDtypeStruct((N, 256), jnp.float32),
            jax.ShapeDtypeStruct((N, HID), jnp.float32),
            jax.ShapeDtypeStruct((1, HID), jnp.float32),
        ],
    )(p0, p1, h, sc)


def _tc_final_edge(F, hsrc, w1p, b1, w2, lin2t):
    def body(f_ref, h_ref, w1_ref, b1_ref, w2_ref, l2_ref, o_ref):
        hid = jnp.maximum(
            jnp.dot(f_ref[...], w1_ref[...], preferred_element_type=jnp.float32)
            + b1_ref[...], 0.0)
        w = jnp.dot(hid, w2_ref[...], preferred_element_type=jnp.float32)
        m = w * h_ref[:, :HID].astype(jnp.float32)
        q = jnp.dot(m, l2_ref[...], preferred_element_type=jnp.float32)
        t = jnp.sum(f_ref[:, 0:16] * q)

        @pl.when(pl.program_id(0) == 0)
        def _():
            o_ref[...] = jnp.zeros_like(o_ref)

        o_ref[...] += t

    return pl.pallas_call(
        body,
        grid=(EP // BE,),
        in_specs=[
            pl.BlockSpec((BE, 32), lambda i: (i, 0)),
            pl.BlockSpec((BE, 256), lambda i: (i, 0)),
            pl.BlockSpec((32, 100), lambda i: (0, 0)),
            pl.BlockSpec((1, 100), lambda i: (0, 0)),
            pl.BlockSpec((100, HID), lambda i: (0, 0)),
            pl.BlockSpec((HID, 16), lambda i: (0, 0)),
        ],
        out_specs=pl.BlockSpec((1, 1), lambda i: (0, 0)),
        out_shape=jax.ShapeDtypeStruct((1, 1), jnp.float32),
    )(F, hsrc, w1p, b1, w2, lin2t)


def _tc_combine(s, cs, sc2):
    def body(s_ref, cs_ref, sc2_ref, o_ref):
        o_ref[...] = (s_ref[...] * (INV_SQRT_NUM_NEIGH / N)
                      + jnp.dot(cs_ref[...], sc2_ref[...],
                                preferred_element_type=jnp.float32) * (1.0 / N))

    return pl.pallas_call(
        body,
        out_shape=jax.ShapeDtypeStruct((1, 1), jnp.float32),
    )(s, cs, sc2)


def _pad_w1(fcw1):
    return jnp.zeros((32, 100), jnp.float32).at[SH:SH + NB].set(fcw1)


def kernel(x, pos, edge_index, edge_shift, lattice, params):
    pad = EP - E
    src_f = jnp.concatenate([edge_index[0], jnp.zeros((pad,), jnp.int32)])
    dst_p = jnp.concatenate(
        [edge_index[1], jnp.zeros((pad,), jnp.int32)]).reshape(EP // CH, CH)
    shift_p = jnp.pad(edge_shift, ((0, pad), (0, 13)))
    lat_p = jnp.pad(lattice[0], ((0, 13), (0, 13)))
    zn = jnp.zeros((NP, 128), jnp.float32)

    pos_p = jnp.pad(pos, ((0, 0), (0, 125)))
    pp = _sc_gather(pos_p, jnp.concatenate([src_f, dst_p.ravel()]))
    F = _tc_geometry(pp[:EP], pp[EP:], shift_p, lat_p)

    hb = x
    h = x
    cs = None
    for l in range(2):
        di = D_IN if l == 0 else HID
        hsrc = _sc_gather(hb, src_f)
        pa, pb = _tc_layer(F, hsrc, _pad_w1(params['fcw1_%d' % l]),
                           params['fcb1_%d' % l][None, :], params['fcw2_%d' % l],
                           params['lin_%d' % l], di)
        P = _sc_scatter(pa, pb, dst_p, zn)
        hb, h, cs = _tc_node(P[:N], P[NP:NP + N], h, params['sc_%d' % l], di)

    hsrc2 = _sc_gather(hb, src_f)
    lin2t = jnp.pad(params['lin_2'][:, 0].reshape(SH, HID).T, ((0, 0), (0, 16 - SH)))
    s = _tc_final_edge(F, hsrc2, _pad_w1(params['fcw1_2']),
                       params['fcb1_2'][None, :], params['fcw2_2'], lin2t)
    return _tc_combine(s, cs, params['sc_2'])


# speedup vs baseline: 1.3929x; 1.3929x over previous
import functools

import jax
import jax.numpy as jnp
from jax import lax
from jax.experimental import pallas as pl
from jax.experimental.pallas import tpu as pltpu
from jax.experimental.pallas import tpu_sc as plsc

N = 10000
E = 160000
EP = 163840
D_IN = 128
HID = 144
NB = 10
MAX_R = 5.0
INV_SQRT_NUM_NEIGH = 0.25
SH = 9
NP = 10240
BE = 1024
CH = 128


@functools.lru_cache(maxsize=None)
def _mesh():
    return plsc.VectorSubcoreMesh(
        core_axis_name="c", subcore_axis_name="s", num_cores=2, num_subcores=16)


def _sc_gather(table, idx_flat, sc_tiling=False):
    rows = idx_flat.shape[0]
    width = table.shape[1]
    dt = table.dtype
    rowbytes = width * dt.itemsize
    ch = 128 if rowbytes <= 512 else 64
    U = 4
    nch = rows // (32 * ch)
    idxr = idx_flat.reshape(rows // ch, ch)
    cp = (pltpu.CompilerParams(use_tc_tiling_on_sc=False) if sc_tiling
          else None)

    @functools.partial(
        pl.kernel,
        out_type=jax.ShapeDtypeStruct((rows, width), dt),
        mesh=_mesh(),
        scratch_types=[
            pltpu.VMEM((nch, ch), jnp.int32),
            pltpu.VMEM((U * ch, width), dt),
        ] + [pltpu.SemaphoreType.DMA] * (2 * U),
        compiler_params=cp,
    )
    def k(table_hbm, idx_hbm, out_hbm, idx_v, bufs, *sems):
        gsem, wsem = sems[:U], sems[U:]
        wid = lax.axis_index("s") * 2 + lax.axis_index("c")
        pltpu.sync_copy(idx_hbm.at[pl.ds(wid * nch, nch)], idx_v)
        base = wid * nch * ch

        def body(t, carry):
            for u in range(U):
                j = t * U + u
                slot = bufs.at[pl.ds(u * ch, ch)]

                @pl.when(t > 0)
                def _(j=j, slot=slot, u=u):
                    pltpu.make_async_copy(
                        slot, out_hbm.at[pl.ds(base + (j - U) * ch, ch)],
                        wsem[u]).wait()

                pltpu.async_copy(table_hbm.at[idx_v.at[j]], slot, gsem[u])
            for u in range(U):
                j = t * U + u
                slot = bufs.at[pl.ds(u * ch, ch)]
                pltpu.make_async_copy(
                    table_hbm.at[idx_v.at[j]], slot, gsem[u]).wait()
                pltpu.async_copy(
                    slot, out_hbm.at[pl.ds(base + j * ch, ch)], wsem[u])
            return carry

        lax.fori_loop(0, nch // U, body, 0)
        for u in range(U):
            j = nch - U + u
            pltpu.make_async_copy(
                bufs.at[pl.ds(u * ch, ch)],
                out_hbm.at[pl.ds(base + j * ch, ch)], wsem[u]).wait()

    return k(table, idxr)


def _sc_scatter(pa, pb, idx2d, zeros_init):
    nch = EP // (16 * CH)
    rps = NP // 16
    U = 2

    @functools.partial(
        pl.kernel,
        out_type=jax.ShapeDtypeStruct((2 * NP, 128), jnp.float32),
        mesh=_mesh(),
        scratch_types=[
            pltpu.VMEM((nch, CH), jnp.int32),
            pltpu.VMEM((U * CH, 128), jnp.float32),
            pltpu.VMEM_SHARED((NP, 128), jnp.float32),
        ] + [pltpu.SemaphoreType.DMA] * (2 * U),
    )
    def k(pa_hbm, pb_hbm, idx_hbm, z_hbm, out_hbm, idx_v, bufs, acc, *sems):
        rsem, ssem = sems[:U], sems[U:]
        cid = lax.axis_index("c")
        sid = lax.axis_index("s")
        pltpu.sync_copy(z_hbm.at[pl.ds(sid * rps, rps)],
                        acc.at[pl.ds(sid * rps, rps)])
        plsc.subcore_barrier()
        pltpu.sync_copy(idx_hbm.at[pl.ds(sid * nch, nch)], idx_v)
        base = sid * nch * CH

        for p_hbm, c in ((pa_hbm, 0), (pb_hbm, 1)):
            @pl.when(cid == c)
            def _(p_hbm=p_hbm):
                def body(t, carry):
                    for u in range(U):
                        j = t * U + u
                        slot = bufs.at[pl.ds(u * CH, CH)]

                        @pl.when(t > 0)
                        def _(j=j, slot=slot, u=u):
                            pltpu.make_async_copy(
                                slot, acc.at[idx_v.at[j - U]], ssem[u]).wait()

                        pltpu.async_copy(
                            p_hbm.at[pl.ds(base + j * CH, CH)], slot, rsem[u])
                    for u in range(U):
                        j = t * U + u
                        slot = bufs.at[pl.ds(u * CH, CH)]
                        pltpu.make_async_copy(
                            p_hbm.at[pl.ds(base + j * CH, CH)], slot,
                            rsem[u]).wait()
                        pltpu.async_copy(slot, acc.at[idx_v.at[j]], ssem[u],
                                         add=True)
                    return carry

                lax.fori_loop(0, nch // U, body, 0)
                for u in range(U):
                    j = nch - U + u
                    pltpu.make_async_copy(
                        bufs.at[pl.ds(u * CH, CH)], acc.at[idx_v.at[j]],
                        ssem[u]).wait()

        plsc.subcore_barrier()
        pltpu.sync_copy(acc.at[pl.ds(sid * rps, rps)],
                        out_hbm.at[pl.ds(cid * NP + sid * rps, rps)])

    return k(pa, pb, idx2d, zeros_init)


def _tc_geometry(poss, posd, shift_p, lat_p):
    pw = poss.shape[1]

    def body(ps_ref, pd_ref, sf_ref, lat_ref, f_ref):
        i = pl.program_id(0)
        shmm = jnp.dot(sf_ref[...], lat_ref[...],
                       preferred_element_type=jnp.float32)
        vs = pd_ref[:, :16] - ps_ref[:, :16]
        vx = vs[:, 0:1] + shmm[:, 0:1]
        vy = vs[:, 1:2] + shmm[:, 1:2]
        vz = vs[:, 2:3] + shmm[:, 2:3]
        r = jnp.sqrt(vx * vx + vy * vy + vz * vz + 1e-9)
        xx, yy, zz = vx / r, vy / r, vz / r
        s3 = 3.0 ** 0.5
        s15 = 15.0 ** 0.5
        sh = jnp.concatenate([
            jnp.ones_like(xx), s3 * xx, s3 * yy, s3 * zz,
            s15 * xx * yy, s15 * yy * zz,
            ((5.0 ** 0.5) / 2.0) * (3.0 * zz * zz - 1.0),
            s15 * xx * zz, (s15 / 2.0) * (xx * xx - yy * yy),
        ], axis=1)
        eidx = i * BE + lax.broadcasted_iota(jnp.int32, (BE, 1), 0)
        sh = sh * (eidx < E).astype(jnp.float32)
        step = MAX_R / (NB + 1)
        vals = (lax.broadcasted_iota(jnp.int32, (1, NB), 1).astype(jnp.float32)
                + 1.0) * step
        diff = (r - vals) / step
        emb = (jnp.cos(jnp.pi / 2.0 * diff)
               * ((diff > -1.0) & (diff < 1.0)).astype(jnp.float32)
               * (NB ** 0.5))
        f_ref[...] = jnp.concatenate(
            [sh, emb, jnp.zeros((BE, 32 - SH - NB), jnp.float32)], axis=1)

    return pl.pallas_call(
        body,
        grid=(EP // BE,),
        in_specs=[
            pl.BlockSpec((BE, pw), lambda i: (i, 0)),
            pl.BlockSpec((BE, pw), lambda i: (i, 0)),
            pl.BlockSpec((BE, 16), lambda i: (i, 0)),
            pl.BlockSpec((16, 16), lambda i: (0, 0)),
        ],
        out_specs=pl.BlockSpec((BE, 32), lambda i: (i, 0)),
        out_shape=jax.ShapeDtypeStruct((EP, 32), jnp.float32),
    )(poss, posd, shift_p, lat_p)


def _tc_layer(F, hsrc, w1p, b1, w2, lin, di):
    tw = hsrc.shape[1]

    def body(f_ref, h_ref, w1_ref, b1_ref, w2_ref, lin_ref, pa_ref, pb_ref):
        hid = jnp.maximum(
            jnp.dot(f_ref[...], w1_ref[...], preferred_element_type=jnp.float32)
            + b1_ref[...], 0.0)
        w = jnp.dot(hid, w2_ref[...], preferred_element_type=jnp.float32)
        m = w * h_ref[:, :di].astype(jnp.float32)
        acc = jnp.zeros((BE, HID), jnp.float32)
        for k in range(SH):
            q = jnp.dot(m, lin_ref[k * di:(k + 1) * di, :],
                        preferred_element_type=jnp.float32)
            acc = acc + f_ref[:, k:k + 1] * q
        pa_ref[...] = acc[:, :128]
        pb_ref[...] = jnp.concatenate(
            [acc[:, 128:], jnp.zeros((BE, 256 - HID), jnp.float32)], axis=1)

    return pl.pallas_call(
        body,
        grid=(EP // BE,),
        in_specs=[
            pl.BlockSpec((BE, 32), lambda i: (i, 0)),
            pl.BlockSpec((BE, tw), lambda i: (i, 0)),
            pl.BlockSpec((32, 100), lambda i: (0, 0)),
            pl.BlockSpec((1, 100), lambda i: (0, 0)),
            pl.BlockSpec((100, di), lambda i: (0, 0)),
            pl.BlockSpec((SH * di, HID), lambda i: (0, 0)),
        ],
        out_specs=[
            pl.BlockSpec((BE, 128), lambda i: (i, 0)),
            pl.BlockSpec((BE, 128), lambda i: (i, 0)),
        ],
        out_shape=[
            jax.ShapeDtypeStruct((EP, 128), jnp.float32),
            jax.ShapeDtypeStruct((EP, 128), jnp.float32),
        ],
    )(F, hsrc, w1p, b1, w2, lin)


def _tc_node(p0, p1, h, sc, di):
    bn = 1000
    hw = h.shape[1]

    def body(p0_ref, p1_ref, h_ref, sc_ref, o_ref, of_ref, cs_ref):
        pre = jnp.concatenate(
            [p0_ref[...], p1_ref[:, :HID - 128]],
            axis=1) * INV_SQRT_NUM_NEIGH + jnp.dot(
            h_ref[:, :di], sc_ref[...], preferred_element_type=jnp.float32)
        hn = jax.nn.gelu(pre)
        o_ref[...] = jnp.concatenate(
            [hn, jnp.zeros((bn, 256 - HID), jnp.float32)], axis=1)
        of_ref[...] = hn

        @pl.when(pl.program_id(0) == 0)
        def _():
            cs_ref[...] = jnp.zeros_like(cs_ref)

        cs_ref[...] += jnp.sum(hn, axis=0, keepdims=True)

    return pl.pallas_call(
        body,
        grid=(N // bn,),
        in_specs=[
            pl.BlockSpec((bn, 128), lambda i: (i, 0)),
            pl.BlockSpec((bn, 128), lambda i: (i, 0)),
            pl.BlockSpec((bn, hw), lambda i: (i, 0)),
            pl.BlockSpec((di, HID), lambda i: (0, 0)),
        ],
        out_specs=[
            pl.BlockSpec((bn, 256), lambda i: (i, 0)),
            pl.BlockSpec((bn, HID), lambda i: (i, 0)),
            pl.BlockSpec((1, HID), lambda i: (0, 0)),
        ],
        out_shape=[
            jax.ShapeDtypeStruct((N, 256), jnp.float32),
            jax.ShapeDtypeStruct((N, HID), jnp.float32),
            jax.ShapeDtypeStruct((1, HID), jnp.float32),
        ],
    )(p0, p1, h, sc)


def _tc_final_edge(F, hsrc, w1p, b1, w2, lin2t):
    def body(f_ref, h_ref, w1_ref, b1_ref, w2_ref, l2_ref, o_ref):
        hid = jnp.maximum(
            jnp.dot(f_ref[...], w1_ref[...], preferred_element_type=jnp.float32)
            + b1_ref[...], 0.0)
        w = jnp.dot(hid, w2_ref[...], preferred_element_type=jnp.float32)
        m = w * h_ref[:, :HID].astype(jnp.float32)
        q = jnp.dot(m, l2_ref[...], preferred_element_type=jnp.float32)
        t = jnp.sum(f_ref[:, 0:16] * q)

        @pl.when(pl.program_id(0) == 0)
        def _():
            o_ref[...] = jnp.zeros_like(o_ref)

        o_ref[...] += t

    return pl.pallas_call(
        body,
        grid=(EP // BE,),
        in_specs=[
            pl.BlockSpec((BE, 32), lambda i: (i, 0)),
            pl.BlockSpec((BE, 256), lambda i: (i, 0)),
            pl.BlockSpec((32, 100), lambda i: (0, 0)),
            pl.BlockSpec((1, 100), lambda i: (0, 0)),
            pl.BlockSpec((100, HID), lambda i: (0, 0)),
            pl.BlockSpec((HID, 16), lambda i: (0, 0)),
        ],
        out_specs=pl.BlockSpec((1, 1), lambda i: (0, 0)),
        out_shape=jax.ShapeDtypeStruct((1, 1), jnp.float32),
    )(F, hsrc, w1p, b1, w2, lin2t)


def _tc_combine(s, cs, sc2):
    def body(s_ref, cs_ref, sc2_ref, o_ref):
        o_ref[...] = (s_ref[...] * (INV_SQRT_NUM_NEIGH / N)
                      + jnp.dot(cs_ref[...], sc2_ref[...],
                                preferred_element_type=jnp.float32) * (1.0 / N))

    return pl.pallas_call(
        body,
        out_shape=jax.ShapeDtypeStruct((1, 1), jnp.float32),
    )(s, cs, sc2)


def _pad_w1(fcw1):
    return jnp.zeros((32, 100), jnp.float32).at[SH:SH + NB].set(fcw1)


def kernel(x, pos, edge_index, edge_shift, lattice, params):
    pad = EP - E
    src_f = jnp.concatenate([edge_index[0], jnp.zeros((pad,), jnp.int32)])
    dst_p = jnp.concatenate(
        [edge_index[1], jnp.zeros((pad,), jnp.int32)]).reshape(EP // CH, CH)
    shift_p = jnp.pad(edge_shift, ((0, pad), (0, 13)))
    lat_p = jnp.pad(lattice[0], ((0, 13), (0, 13)))
    zn = jnp.zeros((NP, 128), jnp.float32)

    pos_p = jnp.pad(pos, ((0, 0), (0, 13)))
    pp = _sc_gather(pos_p, jnp.concatenate([src_f, dst_p.ravel()]),
                    sc_tiling=True)
    F = _tc_geometry(pp[:EP], pp[EP:], shift_p, lat_p)

    hb = x
    h = x
    cs = None
    for l in range(2):
        di = D_IN if l == 0 else HID
        hsrc = _sc_gather(hb, src_f)
        pa, pb = _tc_layer(F, hsrc, _pad_w1(params['fcw1_%d' % l]),
                           params['fcb1_%d' % l][None, :], params['fcw2_%d' % l],
                           params['lin_%d' % l], di)
        P = _sc_scatter(pa, pb, dst_p, zn)
        hb, h, cs = _tc_node(P[:N], P[NP:NP + N], h, params['sc_%d' % l], di)

    hsrc2 = _sc_gather(hb, src_f)
    lin2t = jnp.pad(params['lin_2'][:, 0].reshape(SH, HID).T, ((0, 0), (0, 16 - SH)))
    s = _tc_final_edge(F, hsrc2, _pad_w1(params['fcw1_2']),
                       params['fcb1_2'][None, :], params['fcw2_2'], lin2t)
    return _tc_combine(s, cs, params['sc_2'])
